# baseline (speedup 1.0000x reference)
import functools

import jax
import jax.numpy as jnp
from jax import lax
from jax.experimental import pallas as pl
from jax.experimental.pallas import tpu as pltpu
from jax.experimental.pallas import tpu_sc as plsc

F32 = jnp.float32
H = 128
NODE_TYPES = 9


def _ln(v, g, b):
    m = jnp.mean(v, axis=-1, keepdims=True)
    var = jnp.mean((v - m) ** 2, axis=-1, keepdims=True)
    return (v - m) / jnp.sqrt(var + 1e-5) * g + b


def _colnorm(f):
    m = jnp.mean(f, axis=0, keepdims=True)
    v = jnp.mean((f - m) ** 2, axis=0, keepdims=True)
    s = jnp.maximum(jnp.sqrt(v), 1e-8)
    return (f - m) / s


def _stats_body(x_ref, y_ref, n_ref, na_ref, tgt_ref):
    x = x_ref[...]
    yv = y_ref[...]
    nn = n_ref[...]
    vel_raw = yv[:, :3] - x[:, :3]
    vel = jnp.where(nn == 1, vel_raw, 0.0)
    ids = lax.broadcasted_iota(jnp.int32, (x.shape[0], NODE_TYPES), 1)
    oh = (ids == nn).astype(F32)
    feats = jnp.concatenate([vel, oh], axis=1)
    na_ref[...] = _colnorm(feats)
    tgt = jnp.concatenate([vel_raw, yv[:, 2:3]], axis=1)
    tgt_ref[...] = _colnorm(tgt)


def _encnode_body(na_ref, w1, b1, w2, b2, g, bln, w1s, w1d,
                  h_ref, p_ref, q_ref):
    u = jnp.maximum(jnp.dot(na_ref[...], w1[...],
                            preferred_element_type=F32, precision=lax.Precision.HIGHEST) + b1[...], 0.0)
    v = jnp.dot(u, w2[...], preferred_element_type=F32, precision=lax.Precision.HIGHEST) + b2[...]
    h = _ln(v, g[...], bln[...])
    h_ref[...] = h
    p_ref[...] = jnp.dot(h, w1s[...], preferred_element_type=F32, precision=lax.Precision.HIGHEST)
    q_ref[...] = jnp.dot(h, w1d[...], preferred_element_type=F32, precision=lax.Precision.HIGHEST)


def _encedge_body(ea_ref, w1, b1, w2, b2, g, bln, e_ref):
    u = jnp.maximum(jnp.dot(ea_ref[...], w1[...],
                            preferred_element_type=F32, precision=lax.Precision.HIGHEST) + b1[...], 0.0)
    v = jnp.dot(u, w2[...], preferred_element_type=F32, precision=lax.Precision.HIGHEST) + b2[...]
    e_ref[...] = _ln(v, g[...], bln[...])


def _edgestep_body(e_ref, s_ref, w1e, b1, w2, b2, g, bln, out_ref):
    e = e_ref[...]
    u = jnp.maximum(jnp.dot(e, w1e[...], preferred_element_type=F32, precision=lax.Precision.HIGHEST)
                    + s_ref[...] + b1[...], 0.0)
    v = jnp.dot(u, w2[...], preferred_element_type=F32, precision=lax.Precision.HIGHEST) + b2[...]
    out_ref[...] = e + _ln(v, g[...], bln[...])


def _nodestep_body(h_ref, pa0_ref, pa1_ref, w1h, w1a, b1, w2, b2, g, bln,
                   w1s, w1d, h_out, p_out, q_out):
    h = h_ref[...]
    agg = pa0_ref[...] + pa1_ref[...]
    u = jnp.maximum(jnp.dot(h, w1h[...], preferred_element_type=F32, precision=lax.Precision.HIGHEST)
                    + jnp.dot(agg, w1a[...], preferred_element_type=F32, precision=lax.Precision.HIGHEST)
                    + b1[...], 0.0)
    v = jnp.dot(u, w2[...], preferred_element_type=F32, precision=lax.Precision.HIGHEST) + b2[...]
    hn = h + _ln(v, g[...], bln[...])
    h_out[...] = hn
    p_out[...] = jnp.dot(hn, w1s[...], preferred_element_type=F32, precision=lax.Precision.HIGHEST)
    q_out[...] = jnp.dot(hn, w1d[...], preferred_element_type=F32, precision=lax.Precision.HIGHEST)


def _nodelast_body(h_ref, pa0_ref, pa1_ref, w1h, w1a, b1, w2, b2, g, bln,
                   h_out):
    h = h_ref[...]
    agg = pa0_ref[...] + pa1_ref[...]
    u = jnp.maximum(jnp.dot(h, w1h[...], preferred_element_type=F32, precision=lax.Precision.HIGHEST)
                    + jnp.dot(agg, w1a[...], preferred_element_type=F32, precision=lax.Precision.HIGHEST)
                    + b1[...], 0.0)
    v = jnp.dot(u, w2[...], preferred_element_type=F32, precision=lax.Precision.HIGHEST) + b2[...]
    h_out[...] = h + _ln(v, g[...], bln[...])


def _dec_body(h_ref, w1, b1, w2, b2, out_ref):
    u = jnp.maximum(jnp.dot(h_ref[...], w1[...],
                            preferred_element_type=F32, precision=lax.Precision.HIGHEST) + b1[...], 0.0)
    out_ref[...] = jnp.dot(u, w2[...], preferred_element_type=F32, precision=lax.Precision.HIGHEST) + b2[...]


def _row(b, c):
    return pl.BlockSpec((b, c), lambda i: (i, 0))


def _bcast(r, c):
    return pl.BlockSpec((r, c), lambda i: (0, 0))


_NC, _NS = 2, 16
_NW = _NC * _NS


@functools.cache
def _make_gather(n_nodes, n_edges):
    epw = n_edges // _NW
    ch = 80
    nchunk = epw // ch
    npair = (nchunk + 1) // 2
    mesh = plsc.VectorSubcoreMesh(core_axis_name="c", subcore_axis_name="s",
                                  num_cores=_NC, num_subcores=_NS)

    @functools.partial(
        pl.kernel,
        out_type=jax.ShapeDtypeStruct((n_edges, H), F32),
        mesh=mesh,
        scratch_types=[
            pltpu.VMEM((nchunk, ch), jnp.int32),
            pltpu.VMEM((nchunk, ch), jnp.int32),
            pltpu.VMEM((ch, H), F32),
            pltpu.VMEM((ch, H), F32),
            pltpu.VMEM((ch, H), F32),
            pltpu.VMEM((ch, H), F32),
            pltpu.VMEM((ch, H), F32),
            pltpu.VMEM((ch, H), F32),
            pltpu.SemaphoreType.DMA,
            pltpu.SemaphoreType.DMA,
            pltpu.SemaphoreType.DMA,
            pltpu.SemaphoreType.DMA,
            pltpu.SemaphoreType.DMA,
            pltpu.SemaphoreType.DMA,
        ],
    )
    def gather_k(p_hbm, q_hbm, src3_hbm, dst3_hbm, out_hbm,
                 idxs, idxd, bufp0, bufp1, bufq0, bufq1, bufs0, bufs1,
                 semp0, semp1, semq0, semq1, semo0, semo1):
        wid = lax.axis_index("s") * _NC + lax.axis_index("c")
        base0 = wid * epw
        bufp = (bufp0, bufp1)
        bufq = (bufq0, bufq1)
        bufs = (bufs0, bufs1)
        semp = (semp0, semp1)
        semq = (semq0, semq1)
        semo = (semo0, semo1)

        pltpu.sync_copy(src3_hbm.at[wid], idxs)
        pltpu.sync_copy(dst3_hbm.at[wid], idxd)

        def g_start(ci, par):
            pltpu.async_copy(p_hbm.at[idxs.at[ci]], bufp[par], semp[par])
            pltpu.async_copy(q_hbm.at[idxd.at[ci]], bufq[par], semq[par])

        def g_wait(ci, par):
            pltpu.make_async_copy(p_hbm.at[idxs.at[ci]], bufp[par],
                                  semp[par]).wait()
            pltpu.make_async_copy(q_hbm.at[idxd.at[ci]], bufq[par],
                                  semq[par]).wait()

        def vadd(par):
            bp, bq, bs = bufp[par], bufq[par], bufs[par]

            def row(r, acc):
                for c8 in range(H // 16):
                    sl = pl.ds(c8 * 16, 16)
                    bs[r, sl] = bp[r, sl] + bq[r, sl]
                return acc

            lax.fori_loop(0, ch, row, 0)

        def st_start(ci, par):
            pltpu.async_copy(bufs[par],
                             out_hbm.at[pl.ds(base0 + ci * ch, ch)],
                             semo[par])

        def st_wait(par):
            pltpu.make_async_copy(bufs[par], out_hbm.at[pl.ds(0, ch)],
                                  semo[par]).wait()

        g_start(0, 0)

        def pair(i, carry):
            c0 = 2 * i
            c1 = c0 + 1
            c2 = c0 + 2

            @pl.when(jnp.logical_and(c1 < nchunk, i >= 1))
            def _():
                st_wait(1)

            @pl.when(c1 < nchunk)
            def _():
                g_start(c1, 1)

            g_wait(c0, 0)
            vadd(0)
            st_start(c0, 0)

            @pl.when(c1 < nchunk)
            def _():
                g_wait(c1, 1)
                vadd(1)
                st_start(c1, 1)

            @pl.when(c2 < nchunk)
            def _():
                st_wait(0)
                g_start(c2, 0)

            return carry

        lax.fori_loop(0, npair, pair, 0)
        st_wait(0)
        if nchunk > 1:
            st_wait(1)

    return gather_k


@functools.cache
def _make_scatter(n_nodes, n_edges):
    epw = n_edges // _NW
    ch = 80
    nchunk = epw // ch
    rpt = (n_nodes // _NS) // 8 * 8
    tail = n_nodes - rpt * _NS
    mesh = plsc.VectorSubcoreMesh(core_axis_name="c", subcore_axis_name="s",
                                  num_cores=_NC, num_subcores=_NS)

    npair = (nchunk + 1) // 2

    @functools.partial(
        pl.kernel,
        out_type=jax.ShapeDtypeStruct((_NC, n_nodes, H), F32),
        mesh=mesh,
        scratch_types=[
            pltpu.VMEM((nchunk, ch), jnp.int32),
            pltpu.VMEM((ch, H), F32),
            pltpu.VMEM((ch, H), F32),
            pltpu.VMEM((48, H), F32),
            pltpu.VMEM_SHARED((n_nodes, H), F32),
            pltpu.SemaphoreType.DMA,
            pltpu.SemaphoreType.DMA,
        ],
    )
    def scatter_k(e_hbm, dst3_hbm, out_hbm, idx, buf0, buf1, zbuf, acc,
                  sem0, sem1):
        cid = lax.axis_index("c")
        sid = lax.axis_index("s")
        wid = sid * _NC + cid
        row0 = pl.multiple_of(sid * rpt, 8)
        buf = (buf0, buf1)
        sem = (sem0, sem1)
        base0 = wid * epw

        pltpu.sync_copy(dst3_hbm.at[wid], idx)

        def zrow(r, carry):
            for c8 in range(H // 16):
                zbuf[r, pl.ds(c8 * 16, 16)] = jnp.zeros((16,), F32)
            return carry

        lax.fori_loop(0, 48, zrow, 0)

        def zcp(j, carry):
            pltpu.sync_copy(zbuf, acc.at[pl.ds(row0 + j * 48, 48)])
            return carry

        lax.fori_loop(0, rpt // 48, zcp, 0)

        @pl.when(sid == 0)
        def _():
            pltpu.sync_copy(zbuf.at[pl.ds(0, tail)],
                            acc.at[pl.ds(rpt * _NS, tail)])

        plsc.subcore_barrier()

        def e_start(ci, par):
            pltpu.async_copy(e_hbm.at[pl.ds(base0 + ci * ch, ch)],
                             buf[par], sem[par])

        def e_wait(par):
            pltpu.make_async_copy(e_hbm.at[pl.ds(0, ch)], buf[par],
                                  sem[par]).wait()

        def scat(ci, par):
            pltpu.sync_copy(buf[par], acc.at[idx.at[ci]], add=True)

        e_start(0, 0)

        def pair(i, carry):
            c0 = 2 * i
            c1 = c0 + 1
            c2 = c0 + 2

            @pl.when(c1 < nchunk)
            def _():
                e_start(c1, 1)

            e_wait(0)
            scat(c0, 0)

            @pl.when(c2 < nchunk)
            def _():
                e_start(c2, 0)

            @pl.when(c1 < nchunk)
            def _():
                e_wait(1)
                scat(c1, 1)

            return carry

        lax.fori_loop(0, npair, pair, 0)
        plsc.subcore_barrier()
        pltpu.sync_copy(acc.at[pl.ds(row0, rpt)],
                        out_hbm.at[cid, pl.ds(row0, rpt)])

        @pl.when(sid == 0)
        def _():
            pltpu.sync_copy(acc.at[pl.ds(rpt * _NS, tail)],
                            out_hbm.at[cid, pl.ds(rpt * _NS, tail)])

    return scatter_k


def kernel(x, y, n, edge_index, edge_attr, params):
    n_nodes = x.shape[0]
    n_edges = edge_attr.shape[0]
    src = edge_index[0]
    dst = edge_index[1]
    src3 = src.reshape(_NW, -1, 80)
    dst3 = dst.reshape(_NW, -1, 80)
    n2 = n.reshape(n_nodes, 1).astype(jnp.int32)

    bn = 2000
    gn = n_nodes // bn
    be = 2560
    ge = n_edges // be

    def rs(v):
        return v.reshape(1, -1)

    enc_n = params['enc_node']
    enc_e = params['enc_edge']
    mp = params['mp']
    dec = params['dec']

    na, tgt = pl.pallas_call(
        _stats_body,
        out_shape=(jax.ShapeDtypeStruct((n_nodes, 3 + NODE_TYPES), F32),
                   jax.ShapeDtypeStruct((n_nodes, 4), F32)),
    )(x, y, n2)

    w1s0 = mp[0]['edge']['W1'][H:2 * H]
    w1d0 = mp[0]['edge']['W1'][2 * H:]
    h, p_tab, q_tab = pl.pallas_call(
        _encnode_body,
        grid=(gn,),
        in_specs=[_row(bn, 3 + NODE_TYPES), _bcast(3 + NODE_TYPES, H),
                  _bcast(1, H), _bcast(H, H), _bcast(1, H), _bcast(1, H),
                  _bcast(1, H), _bcast(H, H), _bcast(H, H)],
        out_specs=(_row(bn, H), _row(bn, H), _row(bn, H)),
        out_shape=(jax.ShapeDtypeStruct((n_nodes, H), F32),) * 3,
    )(na, enc_n['W1'], rs(enc_n['b1']), enc_n['W2'], rs(enc_n['b2']),
      rs(enc_n['g']), rs(enc_n['bln']), w1s0, w1d0)

    e = pl.pallas_call(
        _encedge_body,
        grid=(ge,),
        in_specs=[_row(be, 4), _bcast(4, H), _bcast(1, H), _bcast(H, H),
                  _bcast(1, H), _bcast(1, H), _bcast(1, H)],
        out_specs=_row(be, H),
        out_shape=jax.ShapeDtypeStruct((n_edges, H), F32),
    )(edge_attr, enc_e['W1'], rs(enc_e['b1']), enc_e['W2'],
      rs(enc_e['b2']), rs(enc_e['g']), rs(enc_e['bln']))

    gather_k = _make_gather(n_nodes, n_edges)
    scatter_k = _make_scatter(n_nodes, n_edges)

    n_steps = len(mp)
    for i in range(n_steps):
        blk = mp[i]
        ew = blk['edge']
        nw = blk['node']
        s = gather_k(p_tab, q_tab, src3, dst3)
        e = pl.pallas_call(
            _edgestep_body,
            grid=(ge,),
            in_specs=[_row(be, H), _row(be, H), _bcast(H, H), _bcast(1, H),
                      _bcast(H, H), _bcast(1, H), _bcast(1, H), _bcast(1, H)],
            out_specs=_row(be, H),
            out_shape=jax.ShapeDtypeStruct((n_edges, H), F32),
        )(e, s, ew['W1'][:H], rs(ew['b1']), ew['W2'], rs(ew['b2']),
          rs(ew['g']), rs(ew['bln']))
        parts = scatter_k(e, dst3)
        p0, p1 = parts[0], parts[1]
        if i + 1 < n_steps:
            w1s = mp[i + 1]['edge']['W1'][H:2 * H]
            w1d = mp[i + 1]['edge']['W1'][2 * H:]
            h, p_tab, q_tab = pl.pallas_call(
                _nodestep_body,
                grid=(gn,),
                in_specs=[_row(bn, H), _row(bn, H), _row(bn, H),
                          _bcast(H, H), _bcast(H, H), _bcast(1, H),
                          _bcast(H, H), _bcast(1, H), _bcast(1, H),
                          _bcast(1, H), _bcast(H, H), _bcast(H, H)],
                out_specs=(_row(bn, H),) * 3,
                out_shape=(jax.ShapeDtypeStruct((n_nodes, H), F32),) * 3,
            )(h, p0, p1, nw['W1'][:H], nw['W1'][H:], rs(nw['b1']),
              nw['W2'], rs(nw['b2']), rs(nw['g']), rs(nw['bln']), w1s, w1d)
        else:
            h = pl.pallas_call(
                _nodelast_body,
                grid=(gn,),
                in_specs=[_row(bn, H), _row(bn, H), _row(bn, H),
                          _bcast(H, H), _bcast(H, H), _bcast(1, H),
                          _bcast(H, H), _bcast(1, H), _bcast(1, H),
                          _bcast(1, H)],
                out_specs=_row(bn, H),
                out_shape=jax.ShapeDtypeStruct((n_nodes, H), F32),
            )(h, p0, p1, nw['W1'][:H], nw['W1'][H:], rs(nw['b1']),
              nw['W2'], rs(nw['b2']), rs(nw['g']), rs(nw['bln']))

    pred = pl.pallas_call(
        _dec_body,
        grid=(gn,),
        in_specs=[_row(bn, H), _bcast(H, H), _bcast(1, H), _bcast(H, 4),
                  _bcast(1, 4)],
        out_specs=_row(bn, 4),
        out_shape=jax.ShapeDtypeStruct((n_nodes, 4), F32),
    )(h, dec['W1'], rs(dec['b1']), dec['W2'], rs(dec['b2']))

    return (pred, tgt)


# speedup vs baseline: 1.7803x; 1.7803x over previous
import functools

import jax
import jax.numpy as jnp
from jax import lax
from jax.experimental import pallas as pl
from jax.experimental.pallas import tpu as pltpu
from jax.experimental.pallas import tpu_sc as plsc

F32 = jnp.float32
H = 128
NODE_TYPES = 9


def _ln(v, g, b):
    m = jnp.mean(v, axis=-1, keepdims=True)
    var = jnp.mean((v - m) ** 2, axis=-1, keepdims=True)
    return (v - m) / jnp.sqrt(var + 1e-5) * g + b


def _colnorm(f):
    m = jnp.mean(f, axis=0, keepdims=True)
    v = jnp.mean((f - m) ** 2, axis=0, keepdims=True)
    s = jnp.maximum(jnp.sqrt(v), 1e-8)
    return (f - m) / s


def _stats_body(x_ref, y_ref, n_ref, na_ref, tgt_ref):
    x = x_ref[...]
    yv = y_ref[...]
    nn = n_ref[...]
    vel_raw = yv[:, :3] - x[:, :3]
    vel = jnp.where(nn == 1, vel_raw, 0.0)
    ids = lax.broadcasted_iota(jnp.int32, (x.shape[0], NODE_TYPES), 1)
    oh = (ids == nn).astype(F32)
    feats = jnp.concatenate([vel, oh], axis=1)
    na_ref[...] = _colnorm(feats)
    tgt = jnp.concatenate([vel_raw, yv[:, 2:3]], axis=1)
    tgt_ref[...] = _colnorm(tgt)


def _encnode_body(na_ref, w1, b1, w2, b2, g, bln, w1s, w1d,
                  h_ref, p_ref, q_ref):
    u = jnp.maximum(jnp.dot(na_ref[...], w1[...],
                            preferred_element_type=F32) + b1[...], 0.0)
    v = jnp.dot(u, w2[...], preferred_element_type=F32) + b2[...]
    h = _ln(v, g[...], bln[...])
    h_ref[...] = h
    p_ref[...] = jnp.dot(h, w1s[...], preferred_element_type=F32)
    q_ref[...] = jnp.dot(h, w1d[...], preferred_element_type=F32)


def _encedge_body(ea_ref, w1, b1, w2, b2, g, bln, e_ref):
    u = jnp.maximum(jnp.dot(ea_ref[...], w1[...],
                            preferred_element_type=F32) + b1[...], 0.0)
    v = jnp.dot(u, w2[...], preferred_element_type=F32) + b2[...]
    e_ref[...] = _ln(v, g[...], bln[...])


def _edgestep_body(e_ref, s_ref, w1e, b1, w2, b2, g, bln, out_ref):
    e = e_ref[...]
    u = jnp.maximum(jnp.dot(e, w1e[...], preferred_element_type=F32)
                    + s_ref[...] + b1[...], 0.0)
    v = jnp.dot(u, w2[...], preferred_element_type=F32) + b2[...]
    out_ref[...] = e + _ln(v, g[...], bln[...])


def _nodestep_body(h_ref, pa0_ref, pa1_ref, w1h, w1a, b1, w2, b2, g, bln,
                   w1s, w1d, h_out, p_out, q_out):
    h = h_ref[...]
    agg = pa0_ref[...] + pa1_ref[...]
    u = jnp.maximum(jnp.dot(h, w1h[...], preferred_element_type=F32)
                    + jnp.dot(agg, w1a[...], preferred_element_type=F32)
                    + b1[...], 0.0)
    v = jnp.dot(u, w2[...], preferred_element_type=F32) + b2[...]
    hn = h + _ln(v, g[...], bln[...])
    h_out[...] = hn
    p_out[...] = jnp.dot(hn, w1s[...], preferred_element_type=F32)
    q_out[...] = jnp.dot(hn, w1d[...], preferred_element_type=F32)


def _nodelast_body(h_ref, pa0_ref, pa1_ref, w1h, w1a, b1, w2, b2, g, bln,
                   h_out):
    h = h_ref[...]
    agg = pa0_ref[...] + pa1_ref[...]
    u = jnp.maximum(jnp.dot(h, w1h[...], preferred_element_type=F32)
                    + jnp.dot(agg, w1a[...], preferred_element_type=F32)
                    + b1[...], 0.0)
    v = jnp.dot(u, w2[...], preferred_element_type=F32) + b2[...]
    h_out[...] = h + _ln(v, g[...], bln[...])


def _dec_body(h_ref, w1, b1, w2, b2, out_ref):
    u = jnp.maximum(jnp.dot(h_ref[...], w1[...],
                            preferred_element_type=F32) + b1[...], 0.0)
    out_ref[...] = jnp.dot(u, w2[...], preferred_element_type=F32) + b2[...]


def _row(b, c):
    return pl.BlockSpec((b, c), lambda i: (i, 0))


def _bcast(r, c):
    return pl.BlockSpec((r, c), lambda i: (0, 0))


_NC, _NS = 2, 16
_NW = _NC * _NS


@functools.cache
def _make_gather(n_nodes, n_edges):
    epw = n_edges // _NW
    ch = 80
    nchunk = epw // ch
    npair = (nchunk + 1) // 2
    mesh = plsc.VectorSubcoreMesh(core_axis_name="c", subcore_axis_name="s",
                                  num_cores=_NC, num_subcores=_NS)

    @functools.partial(
        pl.kernel,
        out_type=jax.ShapeDtypeStruct((n_edges, H), F32),
        mesh=mesh,
        scratch_types=[
            pltpu.VMEM((nchunk, ch), jnp.int32),
            pltpu.VMEM((nchunk, ch), jnp.int32),
            pltpu.VMEM((ch, H), F32),
            pltpu.VMEM((ch, H), F32),
            pltpu.VMEM((ch, H), F32),
            pltpu.VMEM((ch, H), F32),
            pltpu.VMEM((ch, H), F32),
            pltpu.VMEM((ch, H), F32),
            pltpu.SemaphoreType.DMA,
            pltpu.SemaphoreType.DMA,
            pltpu.SemaphoreType.DMA,
            pltpu.SemaphoreType.DMA,
            pltpu.SemaphoreType.DMA,
            pltpu.SemaphoreType.DMA,
        ],
    )
    def gather_k(p_hbm, q_hbm, src3_hbm, dst3_hbm, out_hbm,
                 idxs, idxd, bufp0, bufp1, bufq0, bufq1, bufs0, bufs1,
                 semp0, semp1, semq0, semq1, semo0, semo1):
        wid = lax.axis_index("s") * _NC + lax.axis_index("c")
        base0 = wid * epw
        bufp = (bufp0, bufp1)
        bufq = (bufq0, bufq1)
        bufs = (bufs0, bufs1)
        semp = (semp0, semp1)
        semq = (semq0, semq1)
        semo = (semo0, semo1)

        pltpu.sync_copy(src3_hbm.at[wid], idxs)
        pltpu.sync_copy(dst3_hbm.at[wid], idxd)

        def g_start(ci, par):
            pltpu.async_copy(p_hbm.at[idxs.at[ci]], bufp[par], semp[par])
            pltpu.async_copy(q_hbm.at[idxd.at[ci]], bufq[par], semq[par])

        def g_wait(ci, par):
            pltpu.make_async_copy(p_hbm.at[idxs.at[ci]], bufp[par],
                                  semp[par]).wait()
            pltpu.make_async_copy(q_hbm.at[idxd.at[ci]], bufq[par],
                                  semq[par]).wait()

        def vadd(par):
            bp, bq, bs = bufp[par], bufq[par], bufs[par]

            def row(r, acc):
                for c8 in range(H // 16):
                    sl = pl.ds(c8 * 16, 16)
                    bs[r, sl] = bp[r, sl] + bq[r, sl]
                return acc

            lax.fori_loop(0, ch, row, 0)

        def st_start(ci, par):
            pltpu.async_copy(bufs[par],
                             out_hbm.at[pl.ds(base0 + ci * ch, ch)],
                             semo[par])

        def st_wait(par):
            pltpu.make_async_copy(bufs[par], out_hbm.at[pl.ds(0, ch)],
                                  semo[par]).wait()

        g_start(0, 0)

        def pair(i, carry):
            c0 = 2 * i
            c1 = c0 + 1
            c2 = c0 + 2

            @pl.when(jnp.logical_and(c1 < nchunk, i >= 1))
            def _():
                st_wait(1)

            @pl.when(c1 < nchunk)
            def _():
                g_start(c1, 1)

            g_wait(c0, 0)
            vadd(0)
            st_start(c0, 0)

            @pl.when(c1 < nchunk)
            def _():
                g_wait(c1, 1)
                vadd(1)
                st_start(c1, 1)

            @pl.when(c2 < nchunk)
            def _():
                st_wait(0)
                g_start(c2, 0)

            return carry

        lax.fori_loop(0, npair, pair, 0)
        st_wait(0)
        if nchunk > 1:
            st_wait(1)

    return gather_k


@functools.cache
def _make_scatter(n_nodes, n_edges):
    epw = n_edges // _NW
    ch = 80
    nchunk = epw // ch
    rpt = (n_nodes // _NS) // 8 * 8
    tail = n_nodes - rpt * _NS
    mesh = plsc.VectorSubcoreMesh(core_axis_name="c", subcore_axis_name="s",
                                  num_cores=_NC, num_subcores=_NS)

    npair = (nchunk + 1) // 2

    @functools.partial(
        pl.kernel,
        out_type=jax.ShapeDtypeStruct((_NC, n_nodes, H), F32),
        mesh=mesh,
        scratch_types=[
            pltpu.VMEM((nchunk, ch), jnp.int32),
            pltpu.VMEM((ch, H), F32),
            pltpu.VMEM((ch, H), F32),
            pltpu.VMEM((48, H), F32),
            pltpu.VMEM_SHARED((n_nodes, H), F32),
            pltpu.SemaphoreType.DMA,
            pltpu.SemaphoreType.DMA,
        ],
    )
    def scatter_k(e_hbm, dst3_hbm, out_hbm, idx, buf0, buf1, zbuf, acc,
                  sem0, sem1):
        cid = lax.axis_index("c")
        sid = lax.axis_index("s")
        wid = sid * _NC + cid
        row0 = pl.multiple_of(sid * rpt, 8)
        buf = (buf0, buf1)
        sem = (sem0, sem1)
        base0 = wid * epw

        pltpu.sync_copy(dst3_hbm.at[wid], idx)

        def zrow(r, carry):
            for c8 in range(H // 16):
                zbuf[r, pl.ds(c8 * 16, 16)] = jnp.zeros((16,), F32)
            return carry

        lax.fori_loop(0, 48, zrow, 0)

        def zcp(j, carry):
            pltpu.sync_copy(zbuf, acc.at[pl.ds(row0 + j * 48, 48)])
            return carry

        lax.fori_loop(0, rpt // 48, zcp, 0)

        @pl.when(sid == 0)
        def _():
            pltpu.sync_copy(zbuf.at[pl.ds(0, tail)],
                            acc.at[pl.ds(rpt * _NS, tail)])

        plsc.subcore_barrier()

        def e_start(ci, par):
            pltpu.async_copy(e_hbm.at[pl.ds(base0 + ci * ch, ch)],
                             buf[par], sem[par])

        def e_wait(par):
            pltpu.make_async_copy(e_hbm.at[pl.ds(0, ch)], buf[par],
                                  sem[par]).wait()

        def scat(ci, par):
            pltpu.sync_copy(buf[par], acc.at[idx.at[ci]], add=True)

        e_start(0, 0)

        def pair(i, carry):
            c0 = 2 * i
            c1 = c0 + 1
            c2 = c0 + 2

            @pl.when(c1 < nchunk)
            def _():
                e_start(c1, 1)

            e_wait(0)
            scat(c0, 0)

            @pl.when(c2 < nchunk)
            def _():
                e_start(c2, 0)

            @pl.when(c1 < nchunk)
            def _():
                e_wait(1)
                scat(c1, 1)

            return carry

        lax.fori_loop(0, npair, pair, 0)
        plsc.subcore_barrier()
        pltpu.sync_copy(acc.at[pl.ds(row0, rpt)],
                        out_hbm.at[cid, pl.ds(row0, rpt)])

        @pl.when(sid == 0)
        def _():
            pltpu.sync_copy(acc.at[pl.ds(rpt * _NS, tail)],
                            out_hbm.at[cid, pl.ds(rpt * _NS, tail)])

    return scatter_k


def kernel(x, y, n, edge_index, edge_attr, params):
    n_nodes = x.shape[0]
    n_edges = edge_attr.shape[0]
    src = edge_index[0]
    dst = edge_index[1]
    src3 = src.reshape(_NW, -1, 80)
    dst3 = dst.reshape(_NW, -1, 80)
    n2 = n.reshape(n_nodes, 1).astype(jnp.int32)

    bn = 2000
    gn = n_nodes // bn
    be = 4000
    ge = n_edges // be

    def rs(v):
        return v.reshape(1, -1)

    enc_n = params['enc_node']
    enc_e = params['enc_edge']
    mp = params['mp']
    dec = params['dec']

    na, tgt = pl.pallas_call(
        _stats_body,
        out_shape=(jax.ShapeDtypeStruct((n_nodes, 3 + NODE_TYPES), F32),
                   jax.ShapeDtypeStruct((n_nodes, 4), F32)),
    )(x, y, n2)

    w1s0 = mp[0]['edge']['W1'][H:2 * H]
    w1d0 = mp[0]['edge']['W1'][2 * H:]
    h, p_tab, q_tab = pl.pallas_call(
        _encnode_body,
        grid=(gn,),
        in_specs=[_row(bn, 3 + NODE_TYPES), _bcast(3 + NODE_TYPES, H),
                  _bcast(1, H), _bcast(H, H), _bcast(1, H), _bcast(1, H),
                  _bcast(1, H), _bcast(H, H), _bcast(H, H)],
        out_specs=(_row(bn, H), _row(bn, H), _row(bn, H)),
        out_shape=(jax.ShapeDtypeStruct((n_nodes, H), F32),) * 3,
    )(na, enc_n['W1'], rs(enc_n['b1']), enc_n['W2'], rs(enc_n['b2']),
      rs(enc_n['g']), rs(enc_n['bln']), w1s0, w1d0)

    e = pl.pallas_call(
        _encedge_body,
        grid=(ge,),
        in_specs=[_row(be, 4), _bcast(4, H), _bcast(1, H), _bcast(H, H),
                  _bcast(1, H), _bcast(1, H), _bcast(1, H)],
        out_specs=_row(be, H),
        out_shape=jax.ShapeDtypeStruct((n_edges, H), F32),
    )(edge_attr, enc_e['W1'], rs(enc_e['b1']), enc_e['W2'],
      rs(enc_e['b2']), rs(enc_e['g']), rs(enc_e['bln']))

    gather_k = _make_gather(n_nodes, n_edges)
    scatter_k = _make_scatter(n_nodes, n_edges)

    n_steps = len(mp)
    for i in range(n_steps):
        blk = mp[i]
        ew = blk['edge']
        nw = blk['node']
        s = gather_k(p_tab, q_tab, src3, dst3)
        e = pl.pallas_call(
            _edgestep_body,
            grid=(ge,),
            in_specs=[_row(be, H), _row(be, H), _bcast(H, H), _bcast(1, H),
                      _bcast(H, H), _bcast(1, H), _bcast(1, H), _bcast(1, H)],
            out_specs=_row(be, H),
            out_shape=jax.ShapeDtypeStruct((n_edges, H), F32),
        )(e, s, ew['W1'][:H], rs(ew['b1']), ew['W2'], rs(ew['b2']),
          rs(ew['g']), rs(ew['bln']))
        parts = scatter_k(e, dst3)
        p0, p1 = parts[0], parts[1]
        if i + 1 < n_steps:
            w1s = mp[i + 1]['edge']['W1'][H:2 * H]
            w1d = mp[i + 1]['edge']['W1'][2 * H:]
            h, p_tab, q_tab = pl.pallas_call(
                _nodestep_body,
                grid=(gn,),
                in_specs=[_row(bn, H), _row(bn, H), _row(bn, H),
                          _bcast(H, H), _bcast(H, H), _bcast(1, H),
                          _bcast(H, H), _bcast(1, H), _bcast(1, H),
                          _bcast(1, H), _bcast(H, H), _bcast(H, H)],
                out_specs=(_row(bn, H),) * 3,
                out_shape=(jax.ShapeDtypeStruct((n_nodes, H), F32),) * 3,
            )(h, p0, p1, nw['W1'][:H], nw['W1'][H:], rs(nw['b1']),
              nw['W2'], rs(nw['b2']), rs(nw['g']), rs(nw['bln']), w1s, w1d)
        else:
            h = pl.pallas_call(
                _nodelast_body,
                grid=(gn,),
                in_specs=[_row(bn, H), _row(bn, H), _row(bn, H),
                          _bcast(H, H), _bcast(H, H), _bcast(1, H),
                          _bcast(H, H), _bcast(1, H), _bcast(1, H),
                          _bcast(1, H)],
                out_specs=_row(bn, H),
                out_shape=jax.ShapeDtypeStruct((n_nodes, H), F32),
            )(h, p0, p1, nw['W1'][:H], nw['W1'][H:], rs(nw['b1']),
              nw['W2'], rs(nw['b2']), rs(nw['g']), rs(nw['bln']))

    pred = pl.pallas_call(
        _dec_body,
        grid=(gn,),
        in_specs=[_row(bn, H), _bcast(H, H), _bcast(1, H), _bcast(H, 4),
                  _bcast(1, 4)],
        out_specs=_row(bn, 4),
        out_shape=jax.ShapeDtypeStruct((n_nodes, 4), F32),
    )(h, dec['W1'], rs(dec['b1']), dec['W2'], rs(dec['b2']))

    return (pred, tgt)


# speedup vs baseline: 1.8750x; 1.0532x over previous
import functools

import jax
import jax.numpy as jnp
from jax import lax
from jax.experimental import pallas as pl
from jax.experimental.pallas import tpu as pltpu
from jax.experimental.pallas import tpu_sc as plsc

F32 = jnp.float32
H = 128
NODE_TYPES = 9


def _ln(v, g, b):
    m = jnp.mean(v, axis=-1, keepdims=True)
    var = jnp.mean((v - m) ** 2, axis=-1, keepdims=True)
    return (v - m) / jnp.sqrt(var + 1e-5) * g + b


def _colnorm(f):
    m = jnp.mean(f, axis=0, keepdims=True)
    v = jnp.mean((f - m) ** 2, axis=0, keepdims=True)
    s = jnp.maximum(jnp.sqrt(v), 1e-8)
    return (f - m) / s


def _stats_body(x_ref, y_ref, n_ref, na_ref, tgt_ref):
    x = x_ref[...]
    yv = y_ref[...]
    nn = n_ref[...]
    vel_raw = yv[:, :3] - x[:, :3]
    vel = jnp.where(nn == 1, vel_raw, 0.0)
    ids = lax.broadcasted_iota(jnp.int32, (x.shape[0], NODE_TYPES), 1)
    oh = (ids == nn).astype(F32)
    feats = jnp.concatenate([vel, oh], axis=1)
    na_ref[...] = _colnorm(feats)
    tgt = jnp.concatenate([vel_raw, yv[:, 2:3]], axis=1)
    tgt_ref[...] = _colnorm(tgt)


def _encnode_body(na_ref, w1, b1, w2, b2, g, bln, w1s, w1d,
                  h_ref, p_ref, q_ref):
    u = jnp.maximum(jnp.dot(na_ref[...], w1[...],
                            preferred_element_type=F32) + b1[...], 0.0)
    v = jnp.dot(u, w2[...], preferred_element_type=F32) + b2[...]
    h = _ln(v, g[...], bln[...])
    h_ref[...] = h
    p_ref[...] = jnp.dot(h, w1s[...], preferred_element_type=F32)
    q_ref[...] = jnp.dot(h, w1d[...], preferred_element_type=F32)


def _encedge_body(ea_ref, w1, b1, w2, b2, g, bln, e_ref):
    u = jnp.maximum(jnp.dot(ea_ref[...], w1[...],
                            preferred_element_type=F32) + b1[...], 0.0)
    v = jnp.dot(u, w2[...], preferred_element_type=F32) + b2[...]
    e_ref[...] = _ln(v, g[...], bln[...])


def _edgestep_body(e_ref, s_ref, w1e, b1, w2, b2, g, bln, out_ref):
    e = e_ref[...]
    u = jnp.maximum(jnp.dot(e, w1e[...], preferred_element_type=F32)
                    + s_ref[...] + b1[...], 0.0)
    v = jnp.dot(u, w2[...], preferred_element_type=F32) + b2[...]
    out_ref[...] = e + _ln(v, g[...], bln[...])


def _nodestep_body(h_ref, pa0_ref, pa1_ref, w1h, w1a, b1, w2, b2, g, bln,
                   w1s, w1d, h_out, p_out, q_out):
    h = h_ref[...]
    agg = pa0_ref[...] + pa1_ref[...]
    u = jnp.maximum(jnp.dot(h, w1h[...], preferred_element_type=F32)
                    + jnp.dot(agg, w1a[...], preferred_element_type=F32)
                    + b1[...], 0.0)
    v = jnp.dot(u, w2[...], preferred_element_type=F32) + b2[...]
    hn = h + _ln(v, g[...], bln[...])
    h_out[...] = hn
    p_out[...] = jnp.dot(hn, w1s[...], preferred_element_type=F32)
    q_out[...] = jnp.dot(hn, w1d[...], preferred_element_type=F32)


def _nodelast_body(h_ref, pa0_ref, pa1_ref, w1h, w1a, b1, w2, b2, g, bln,
                   h_out):
    h = h_ref[...]
    agg = pa0_ref[...] + pa1_ref[...]
    u = jnp.maximum(jnp.dot(h, w1h[...], preferred_element_type=F32)
                    + jnp.dot(agg, w1a[...], preferred_element_type=F32)
                    + b1[...], 0.0)
    v = jnp.dot(u, w2[...], preferred_element_type=F32) + b2[...]
    h_out[...] = h + _ln(v, g[...], bln[...])


def _dec_body(h_ref, w1, b1, w2, b2, out_ref):
    u = jnp.maximum(jnp.dot(h_ref[...], w1[...],
                            preferred_element_type=F32) + b1[...], 0.0)
    out_ref[...] = jnp.dot(u, w2[...], preferred_element_type=F32) + b2[...]


def _row(b, c):
    return pl.BlockSpec((b, c), lambda i: (i, 0))


def _bcast(r, c):
    return pl.BlockSpec((r, c), lambda i: (0, 0))


_NC, _NS = 2, 16
_NW = _NC * _NS


@functools.cache
def _make_gather(n_nodes, n_edges):
    epw = n_edges // _NW
    ch = 80
    nchunk = epw // ch
    npair = (nchunk + 1) // 2
    mesh = plsc.VectorSubcoreMesh(core_axis_name="c", subcore_axis_name="s",
                                  num_cores=_NC, num_subcores=_NS)

    @functools.partial(
        pl.kernel,
        out_type=jax.ShapeDtypeStruct((n_edges, H), F32),
        mesh=mesh,
        scratch_types=[
            pltpu.VMEM((nchunk, ch), jnp.int32),
            pltpu.VMEM((nchunk, ch), jnp.int32),
            pltpu.VMEM((ch, H), F32),
            pltpu.VMEM((ch, H), F32),
            pltpu.VMEM((ch, H), F32),
            pltpu.VMEM((ch, H), F32),
            pltpu.VMEM((ch, H), F32),
            pltpu.VMEM((ch, H), F32),
            pltpu.SemaphoreType.DMA,
            pltpu.SemaphoreType.DMA,
            pltpu.SemaphoreType.DMA,
            pltpu.SemaphoreType.DMA,
            pltpu.SemaphoreType.DMA,
            pltpu.SemaphoreType.DMA,
        ],
    )
    def gather_k(p_hbm, q_hbm, src3_hbm, dst3_hbm, out_hbm,
                 idxs, idxd, bufp0, bufp1, bufq0, bufq1, bufs0, bufs1,
                 semp0, semp1, semq0, semq1, semo0, semo1):
        wid = lax.axis_index("s") * _NC + lax.axis_index("c")
        base0 = wid * epw
        bufp = (bufp0, bufp1)
        bufq = (bufq0, bufq1)
        bufs = (bufs0, bufs1)
        semp = (semp0, semp1)
        semq = (semq0, semq1)
        semo = (semo0, semo1)

        pltpu.sync_copy(src3_hbm.at[wid], idxs)
        pltpu.sync_copy(dst3_hbm.at[wid], idxd)

        def g_start(ci, par):
            pltpu.async_copy(p_hbm.at[idxs.at[ci]], bufp[par], semp[par])
            pltpu.async_copy(q_hbm.at[idxd.at[ci]], bufq[par], semq[par])

        def g_wait(ci, par):
            pltpu.make_async_copy(p_hbm.at[idxs.at[ci]], bufp[par],
                                  semp[par]).wait()
            pltpu.make_async_copy(q_hbm.at[idxd.at[ci]], bufq[par],
                                  semq[par]).wait()

        def vadd(par):
            bp, bq, bs = bufp[par], bufq[par], bufs[par]

            def row(r, acc):
                for c8 in range(H // 16):
                    sl = pl.ds(c8 * 16, 16)
                    bs[r, sl] = bp[r, sl] + bq[r, sl]
                return acc

            lax.fori_loop(0, ch, row, 0)

        def st_start(ci, par):
            pltpu.async_copy(bufs[par],
                             out_hbm.at[pl.ds(base0 + ci * ch, ch)],
                             semo[par])

        def st_wait(par):
            pltpu.make_async_copy(bufs[par], out_hbm.at[pl.ds(0, ch)],
                                  semo[par]).wait()

        g_start(0, 0)

        def pair(i, carry):
            c0 = 2 * i
            c1 = c0 + 1
            c2 = c0 + 2

            @pl.when(jnp.logical_and(c1 < nchunk, i >= 1))
            def _():
                st_wait(1)

            @pl.when(c1 < nchunk)
            def _():
                g_start(c1, 1)

            g_wait(c0, 0)
            vadd(0)
            st_start(c0, 0)

            @pl.when(c1 < nchunk)
            def _():
                g_wait(c1, 1)
                vadd(1)
                st_start(c1, 1)

            @pl.when(c2 < nchunk)
            def _():
                st_wait(0)
                g_start(c2, 0)

            return carry

        lax.fori_loop(0, npair, pair, 0)
        st_wait(0)
        if nchunk > 1:
            st_wait(1)

    return gather_k


@functools.cache
def _make_scatter(n_nodes, n_edges):
    epw = n_edges // _NW
    ch = 80
    nchunk = epw // ch
    rpt = (n_nodes // _NS) // 8 * 8
    tail = n_nodes - rpt * _NS
    mesh = plsc.VectorSubcoreMesh(core_axis_name="c", subcore_axis_name="s",
                                  num_cores=_NC, num_subcores=_NS)

    npair = (nchunk + 1) // 2

    @functools.partial(
        pl.kernel,
        out_type=jax.ShapeDtypeStruct((_NC, n_nodes, H), F32),
        mesh=mesh,
        scratch_types=[
            pltpu.VMEM((nchunk, ch), jnp.int32),
            pltpu.VMEM((ch, H), F32),
            pltpu.VMEM((ch, H), F32),
            pltpu.VMEM((48, H), F32),
            pltpu.VMEM_SHARED((n_nodes, H), F32),
            pltpu.SemaphoreType.DMA,
            pltpu.SemaphoreType.DMA,
        ],
    )
    def scatter_k(e_hbm, dst3_hbm, out_hbm, idx, buf0, buf1, zbuf, acc,
                  sem0, sem1):
        cid = lax.axis_index("c")
        sid = lax.axis_index("s")
        wid = sid * _NC + cid
        row0 = pl.multiple_of(sid * rpt, 8)
        buf = (buf0, buf1)
        sem = (sem0, sem1)
        base0 = wid * epw

        pltpu.sync_copy(dst3_hbm.at[wid], idx)

        def zrow(r, carry):
            for c8 in range(H // 16):
                zbuf[r, pl.ds(c8 * 16, 16)] = jnp.zeros((16,), F32)
            return carry

        lax.fori_loop(0, 48, zrow, 0)

        def zcp(j, carry):
            pltpu.sync_copy(zbuf, acc.at[pl.ds(row0 + j * 48, 48)])
            return carry

        lax.fori_loop(0, rpt // 48, zcp, 0)

        @pl.when(sid == 0)
        def _():
            pltpu.sync_copy(zbuf.at[pl.ds(0, tail)],
                            acc.at[pl.ds(rpt * _NS, tail)])

        plsc.subcore_barrier()

        def e_start(ci, par):
            pltpu.async_copy(e_hbm.at[pl.ds(base0 + ci * ch, ch)],
                             buf[par], sem[par])

        def e_wait(par):
            pltpu.make_async_copy(e_hbm.at[pl.ds(0, ch)], buf[par],
                                  sem[par]).wait()

        def scat(ci, par):
            pltpu.sync_copy(buf[par], acc.at[idx.at[ci]], add=True)

        e_start(0, 0)

        def pair(i, carry):
            c0 = 2 * i
            c1 = c0 + 1
            c2 = c0 + 2

            @pl.when(c1 < nchunk)
            def _():
                e_start(c1, 1)

            e_wait(0)
            scat(c0, 0)

            @pl.when(c2 < nchunk)
            def _():
                e_start(c2, 0)

            @pl.when(c1 < nchunk)
            def _():
                e_wait(1)
                scat(c1, 1)

            return carry

        lax.fori_loop(0, npair, pair, 0)
        plsc.subcore_barrier()
        pltpu.sync_copy(acc.at[pl.ds(row0, rpt)],
                        out_hbm.at[cid, pl.ds(row0, rpt)])

        @pl.when(sid == 0)
        def _():
            pltpu.sync_copy(acc.at[pl.ds(rpt * _NS, tail)],
                            out_hbm.at[cid, pl.ds(rpt * _NS, tail)])

    return scatter_k


def kernel(x, y, n, edge_index, edge_attr, params):
    n_nodes = x.shape[0]
    n_edges = edge_attr.shape[0]
    src = edge_index[0]
    dst = edge_index[1]
    src3 = src.reshape(_NW, -1, 80)
    dst3 = dst.reshape(_NW, -1, 80)
    n2 = n.reshape(n_nodes, 1).astype(jnp.int32)

    bn = 2000
    gn = n_nodes // bn
    be = 8000
    ge = n_edges // be

    def rs(v):
        return v.reshape(1, -1)

    enc_n = params['enc_node']
    enc_e = params['enc_edge']
    mp = params['mp']
    dec = params['dec']

    na, tgt = pl.pallas_call(
        _stats_body,
        out_shape=(jax.ShapeDtypeStruct((n_nodes, 3 + NODE_TYPES), F32),
                   jax.ShapeDtypeStruct((n_nodes, 4), F32)),
    )(x, y, n2)

    w1s0 = mp[0]['edge']['W1'][H:2 * H]
    w1d0 = mp[0]['edge']['W1'][2 * H:]
    h, p_tab, q_tab = pl.pallas_call(
        _encnode_body,
        grid=(gn,),
        in_specs=[_row(bn, 3 + NODE_TYPES), _bcast(3 + NODE_TYPES, H),
                  _bcast(1, H), _bcast(H, H), _bcast(1, H), _bcast(1, H),
                  _bcast(1, H), _bcast(H, H), _bcast(H, H)],
        out_specs=(_row(bn, H), _row(bn, H), _row(bn, H)),
        out_shape=(jax.ShapeDtypeStruct((n_nodes, H), F32),) * 3,
    )(na, enc_n['W1'], rs(enc_n['b1']), enc_n['W2'], rs(enc_n['b2']),
      rs(enc_n['g']), rs(enc_n['bln']), w1s0, w1d0)

    e = pl.pallas_call(
        _encedge_body,
        grid=(ge,),
        in_specs=[_row(be, 4), _bcast(4, H), _bcast(1, H), _bcast(H, H),
                  _bcast(1, H), _bcast(1, H), _bcast(1, H)],
        out_specs=_row(be, H),
        out_shape=jax.ShapeDtypeStruct((n_edges, H), F32),
    )(edge_attr, enc_e['W1'], rs(enc_e['b1']), enc_e['W2'],
      rs(enc_e['b2']), rs(enc_e['g']), rs(enc_e['bln']))

    gather_k = _make_gather(n_nodes, n_edges)
    scatter_k = _make_scatter(n_nodes, n_edges)

    n_steps = len(mp)
    for i in range(n_steps):
        blk = mp[i]
        ew = blk['edge']
        nw = blk['node']
        s = gather_k(p_tab, q_tab, src3, dst3)
        e = pl.pallas_call(
            _edgestep_body,
            grid=(ge,),
            in_specs=[_row(be, H), _row(be, H), _bcast(H, H), _bcast(1, H),
                      _bcast(H, H), _bcast(1, H), _bcast(1, H), _bcast(1, H)],
            out_specs=_row(be, H),
            out_shape=jax.ShapeDtypeStruct((n_edges, H), F32),
        )(e, s, ew['W1'][:H], rs(ew['b1']), ew['W2'], rs(ew['b2']),
          rs(ew['g']), rs(ew['bln']))
        parts = scatter_k(e, dst3)
        p0, p1 = parts[0], parts[1]
        if i + 1 < n_steps:
            w1s = mp[i + 1]['edge']['W1'][H:2 * H]
            w1d = mp[i + 1]['edge']['W1'][2 * H:]
            h, p_tab, q_tab = pl.pallas_call(
                _nodestep_body,
                grid=(gn,),
                in_specs=[_row(bn, H), _row(bn, H), _row(bn, H),
                          _bcast(H, H), _bcast(H, H), _bcast(1, H),
                          _bcast(H, H), _bcast(1, H), _bcast(1, H),
                          _bcast(1, H), _bcast(H, H), _bcast(H, H)],
                out_specs=(_row(bn, H),) * 3,
                out_shape=(jax.ShapeDtypeStruct((n_nodes, H), F32),) * 3,
            )(h, p0, p1, nw['W1'][:H], nw['W1'][H:], rs(nw['b1']),
              nw['W2'], rs(nw['b2']), rs(nw['g']), rs(nw['bln']), w1s, w1d)
        else:
            h = pl.pallas_call(
                _nodelast_body,
                grid=(gn,),
                in_specs=[_row(bn, H), _row(bn, H), _row(bn, H),
                          _bcast(H, H), _bcast(H, H), _bcast(1, H),
                          _bcast(H, H), _bcast(1, H), _bcast(1, H),
                          _bcast(1, H)],
                out_specs=_row(bn, H),
                out_shape=jax.ShapeDtypeStruct((n_nodes, H), F32),
            )(h, p0, p1, nw['W1'][:H], nw['W1'][H:], rs(nw['b1']),
              nw['W2'], rs(nw['b2']), rs(nw['g']), rs(nw['bln']))

    pred = pl.pallas_call(
        _dec_body,
        grid=(gn,),
        in_specs=[_row(bn, H), _bcast(H, H), _bcast(1, H), _bcast(H, 4),
                  _bcast(1, 4)],
        out_specs=_row(bn, 4),
        out_shape=jax.ShapeDtypeStruct((n_nodes, 4), F32),
    )(h, dec['W1'], rs(dec['b1']), dec['W2'], rs(dec['b2']))

    return (pred, tgt)


# speedup vs baseline: 1.9007x; 1.0137x over previous
import functools

import jax
import jax.numpy as jnp
from jax import lax
from jax.experimental import pallas as pl
from jax.experimental.pallas import tpu as pltpu
from jax.experimental.pallas import tpu_sc as plsc

F32 = jnp.float32
H = 128
NODE_TYPES = 9


def _ln(v, g, b):
    m = jnp.mean(v, axis=-1, keepdims=True)
    var = jnp.mean((v - m) ** 2, axis=-1, keepdims=True)
    return (v - m) / jnp.sqrt(var + 1e-5) * g + b


def _colnorm(f):
    m = jnp.mean(f, axis=0, keepdims=True)
    v = jnp.mean((f - m) ** 2, axis=0, keepdims=True)
    s = jnp.maximum(jnp.sqrt(v), 1e-8)
    return (f - m) / s


def _stats_body(x_ref, y_ref, n_ref, na_ref, tgt_ref):
    x = x_ref[...]
    yv = y_ref[...]
    nn = n_ref[...]
    vel_raw = yv[:, :3] - x[:, :3]
    vel = jnp.where(nn == 1, vel_raw, 0.0)
    ids = lax.broadcasted_iota(jnp.int32, (x.shape[0], NODE_TYPES), 1)
    oh = (ids == nn).astype(F32)
    feats = jnp.concatenate([vel, oh], axis=1)
    na_ref[...] = _colnorm(feats)
    tgt = jnp.concatenate([vel_raw, yv[:, 2:3]], axis=1)
    tgt_ref[...] = _colnorm(tgt)


def _encnode_body(na_ref, w1, b1, w2, b2, g, bln, w1s, w1d,
                  h_ref, p_ref, q_ref):
    u = jnp.maximum(jnp.dot(na_ref[...], w1[...],
                            preferred_element_type=F32) + b1[...], 0.0)
    v = jnp.dot(u, w2[...], preferred_element_type=F32) + b2[...]
    h = _ln(v, g[...], bln[...])
    h_ref[...] = h
    p_ref[...] = jnp.dot(h, w1s[...], preferred_element_type=F32)
    q_ref[...] = jnp.dot(h, w1d[...], preferred_element_type=F32)


def _encedge_body(ea_ref, w1, b1, w2, b2, g, bln, e_ref):
    u = jnp.maximum(jnp.dot(ea_ref[...], w1[...],
                            preferred_element_type=F32) + b1[...], 0.0)
    v = jnp.dot(u, w2[...], preferred_element_type=F32) + b2[...]
    e_ref[...] = _ln(v, g[...], bln[...])


def _edgestep_body(e_ref, s_ref, w1e, b1, w2, b2, g, bln, out_ref):
    e = e_ref[...]
    u = jnp.maximum(jnp.dot(e, w1e[...], preferred_element_type=F32)
                    + s_ref[...] + b1[...], 0.0)
    v = jnp.dot(u, w2[...], preferred_element_type=F32) + b2[...]
    out_ref[...] = e + _ln(v, g[...], bln[...])


def _nodestep_body(h_ref, pa0_ref, pa1_ref, w1h, w1a, b1, w2, b2, g, bln,
                   w1s, w1d, h_out, p_out, q_out):
    h = h_ref[...]
    agg = pa0_ref[...] + pa1_ref[...]
    u = jnp.maximum(jnp.dot(h, w1h[...], preferred_element_type=F32)
                    + jnp.dot(agg, w1a[...], preferred_element_type=F32)
                    + b1[...], 0.0)
    v = jnp.dot(u, w2[...], preferred_element_type=F32) + b2[...]
    hn = h + _ln(v, g[...], bln[...])
    h_out[...] = hn
    p_out[...] = jnp.dot(hn, w1s[...], preferred_element_type=F32)
    q_out[...] = jnp.dot(hn, w1d[...], preferred_element_type=F32)


def _nodelast_body(h_ref, pa0_ref, pa1_ref, w1h, w1a, b1, w2, b2, g, bln,
                   h_out):
    h = h_ref[...]
    agg = pa0_ref[...] + pa1_ref[...]
    u = jnp.maximum(jnp.dot(h, w1h[...], preferred_element_type=F32)
                    + jnp.dot(agg, w1a[...], preferred_element_type=F32)
                    + b1[...], 0.0)
    v = jnp.dot(u, w2[...], preferred_element_type=F32) + b2[...]
    h_out[...] = h + _ln(v, g[...], bln[...])


def _dec_body(h_ref, w1, b1, w2, b2, out_ref):
    u = jnp.maximum(jnp.dot(h_ref[...], w1[...],
                            preferred_element_type=F32) + b1[...], 0.0)
    out_ref[...] = jnp.dot(u, w2[...], preferred_element_type=F32) + b2[...]


def _row(b, c):
    return pl.BlockSpec((b, c), lambda i: (i, 0))


def _bcast(r, c):
    return pl.BlockSpec((r, c), lambda i: (0, 0))


_NC, _NS = 2, 16
_NW = _NC * _NS


@functools.cache
def _make_gather(n_nodes, n_edges):
    epw = n_edges // _NW
    ch = 80
    nchunk = epw // ch
    npair = (nchunk + 1) // 2
    mesh = plsc.VectorSubcoreMesh(core_axis_name="c", subcore_axis_name="s",
                                  num_cores=_NC, num_subcores=_NS)

    @functools.partial(
        pl.kernel,
        out_type=jax.ShapeDtypeStruct((n_edges, H), F32),
        mesh=mesh,
        scratch_types=[
            pltpu.VMEM((nchunk, ch), jnp.int32),
            pltpu.VMEM((nchunk, ch), jnp.int32),
            pltpu.VMEM((ch, H), F32),
            pltpu.VMEM((ch, H), F32),
            pltpu.VMEM((ch, H), F32),
            pltpu.VMEM((ch, H), F32),
            pltpu.VMEM((ch, H), F32),
            pltpu.VMEM((ch, H), F32),
            pltpu.SemaphoreType.DMA,
            pltpu.SemaphoreType.DMA,
            pltpu.SemaphoreType.DMA,
            pltpu.SemaphoreType.DMA,
            pltpu.SemaphoreType.DMA,
            pltpu.SemaphoreType.DMA,
        ],
    )
    def gather_k(p_hbm, q_hbm, src3_hbm, dst3_hbm, out_hbm,
                 idxs, idxd, bufp0, bufp1, bufq0, bufq1, bufs0, bufs1,
                 semp0, semp1, semq0, semq1, semo0, semo1):
        wid = lax.axis_index("s") * _NC + lax.axis_index("c")
        base0 = wid * epw
        bufp = (bufp0, bufp1)
        bufq = (bufq0, bufq1)
        bufs = (bufs0, bufs1)
        semp = (semp0, semp1)
        semq = (semq0, semq1)
        semo = (semo0, semo1)

        pltpu.sync_copy(src3_hbm.at[wid], idxs)
        pltpu.sync_copy(dst3_hbm.at[wid], idxd)

        def g_start(ci, par):
            pltpu.async_copy(p_hbm.at[idxs.at[ci]], bufp[par], semp[par])
            pltpu.async_copy(q_hbm.at[idxd.at[ci]], bufq[par], semq[par])

        def g_wait(ci, par):
            pltpu.make_async_copy(p_hbm.at[idxs.at[ci]], bufp[par],
                                  semp[par]).wait()
            pltpu.make_async_copy(q_hbm.at[idxd.at[ci]], bufq[par],
                                  semq[par]).wait()

        def vadd(par):
            bp, bq, bs = bufp[par], bufq[par], bufs[par]

            def row(r, acc):
                for c8 in range(H // 16):
                    sl = pl.ds(c8 * 16, 16)
                    bs[r, sl] = bp[r, sl] + bq[r, sl]
                return acc

            lax.fori_loop(0, ch, row, 0)

        def st_start(ci, par):
            pltpu.async_copy(bufs[par],
                             out_hbm.at[pl.ds(base0 + ci * ch, ch)],
                             semo[par])

        def st_wait(par):
            pltpu.make_async_copy(bufs[par], out_hbm.at[pl.ds(0, ch)],
                                  semo[par]).wait()

        g_start(0, 0)

        def pair(i, carry):
            c0 = 2 * i
            c1 = c0 + 1
            c2 = c0 + 2

            @pl.when(jnp.logical_and(c1 < nchunk, i >= 1))
            def _():
                st_wait(1)

            @pl.when(c1 < nchunk)
            def _():
                g_start(c1, 1)

            g_wait(c0, 0)
            vadd(0)
            st_start(c0, 0)

            @pl.when(c1 < nchunk)
            def _():
                g_wait(c1, 1)
                vadd(1)
                st_start(c1, 1)

            @pl.when(c2 < nchunk)
            def _():
                st_wait(0)
                g_start(c2, 0)

            return carry

        lax.fori_loop(0, npair, pair, 0)
        st_wait(0)
        if nchunk > 1:
            st_wait(1)

    return gather_k


@functools.cache
def _make_scatter(n_nodes, n_edges):
    epw = n_edges // _NW
    ch = 80
    nchunk = epw // ch
    rpt = (n_nodes // _NS) // 8 * 8
    tail = n_nodes - rpt * _NS
    mesh = plsc.VectorSubcoreMesh(core_axis_name="c", subcore_axis_name="s",
                                  num_cores=_NC, num_subcores=_NS)

    npair = (nchunk + 1) // 2

    @functools.partial(
        pl.kernel,
        out_type=jax.ShapeDtypeStruct((_NC, n_nodes, H), F32),
        mesh=mesh,
        scratch_types=[
            pltpu.VMEM((nchunk, ch), jnp.int32),
            pltpu.VMEM((ch, H), F32),
            pltpu.VMEM((ch, H), F32),
            pltpu.VMEM((48, H), F32),
            pltpu.VMEM_SHARED((n_nodes, H), F32),
            pltpu.SemaphoreType.DMA,
            pltpu.SemaphoreType.DMA,
        ],
    )
    def scatter_k(e_hbm, dst3_hbm, out_hbm, idx, buf0, buf1, zbuf, acc,
                  sem0, sem1):
        cid = lax.axis_index("c")
        sid = lax.axis_index("s")
        wid = sid * _NC + cid
        row0 = pl.multiple_of(sid * rpt, 8)
        buf = (buf0, buf1)
        sem = (sem0, sem1)
        base0 = wid * epw

        pltpu.sync_copy(dst3_hbm.at[wid], idx)

        def zrow(r, carry):
            for c8 in range(H // 16):
                zbuf[r, pl.ds(c8 * 16, 16)] = jnp.zeros((16,), F32)
            return carry

        lax.fori_loop(0, 48, zrow, 0)

        def zcp(j, carry):
            pltpu.sync_copy(zbuf, acc.at[pl.ds(row0 + j * 48, 48)])
            return carry

        lax.fori_loop(0, rpt // 48, zcp, 0)

        @pl.when(sid == 0)
        def _():
            pltpu.sync_copy(zbuf.at[pl.ds(0, tail)],
                            acc.at[pl.ds(rpt * _NS, tail)])

        plsc.subcore_barrier()

        def e_start(ci, par):
            pltpu.async_copy(e_hbm.at[pl.ds(base0 + ci * ch, ch)],
                             buf[par], sem[par])

        def e_wait(par):
            pltpu.make_async_copy(e_hbm.at[pl.ds(0, ch)], buf[par],
                                  sem[par]).wait()

        def scat(ci, par):
            pltpu.sync_copy(buf[par], acc.at[idx.at[ci]], add=True)

        e_start(0, 0)

        def pair(i, carry):
            c0 = 2 * i
            c1 = c0 + 1
            c2 = c0 + 2

            @pl.when(c1 < nchunk)
            def _():
                e_start(c1, 1)

            e_wait(0)
            scat(c0, 0)

            @pl.when(c2 < nchunk)
            def _():
                e_start(c2, 0)

            @pl.when(c1 < nchunk)
            def _():
                e_wait(1)
                scat(c1, 1)

            return carry

        lax.fori_loop(0, npair, pair, 0)
        plsc.subcore_barrier()
        pltpu.sync_copy(acc.at[pl.ds(row0, rpt)],
                        out_hbm.at[cid, pl.ds(row0, rpt)])

        @pl.when(sid == 0)
        def _():
            pltpu.sync_copy(acc.at[pl.ds(rpt * _NS, tail)],
                            out_hbm.at[cid, pl.ds(rpt * _NS, tail)])

    return scatter_k


def kernel(x, y, n, edge_index, edge_attr, params):
    n_nodes = x.shape[0]
    n_edges = edge_attr.shape[0]
    src = edge_index[0]
    dst = edge_index[1]
    src3 = src.reshape(_NW, -1, 80)
    dst3 = dst.reshape(_NW, -1, 80)
    n2 = n.reshape(n_nodes, 1).astype(jnp.int32)

    bn = 2000
    gn = n_nodes // bn
    be = 16000
    ge = n_edges // be

    def rs(v):
        return v.reshape(1, -1)

    enc_n = params['enc_node']
    enc_e = params['enc_edge']
    mp = params['mp']
    dec = params['dec']

    na, tgt = pl.pallas_call(
        _stats_body,
        out_shape=(jax.ShapeDtypeStruct((n_nodes, 3 + NODE_TYPES), F32),
                   jax.ShapeDtypeStruct((n_nodes, 4), F32)),
    )(x, y, n2)

    w1s0 = mp[0]['edge']['W1'][H:2 * H]
    w1d0 = mp[0]['edge']['W1'][2 * H:]
    h, p_tab, q_tab = pl.pallas_call(
        _encnode_body,
        grid=(gn,),
        in_specs=[_row(bn, 3 + NODE_TYPES), _bcast(3 + NODE_TYPES, H),
                  _bcast(1, H), _bcast(H, H), _bcast(1, H), _bcast(1, H),
                  _bcast(1, H), _bcast(H, H), _bcast(H, H)],
        out_specs=(_row(bn, H), _row(bn, H), _row(bn, H)),
        out_shape=(jax.ShapeDtypeStruct((n_nodes, H), F32),) * 3,
    )(na, enc_n['W1'], rs(enc_n['b1']), enc_n['W2'], rs(enc_n['b2']),
      rs(enc_n['g']), rs(enc_n['bln']), w1s0, w1d0)

    e = pl.pallas_call(
        _encedge_body,
        grid=(ge,),
        in_specs=[_row(be, 4), _bcast(4, H), _bcast(1, H), _bcast(H, H),
                  _bcast(1, H), _bcast(1, H), _bcast(1, H)],
        out_specs=_row(be, H),
        out_shape=jax.ShapeDtypeStruct((n_edges, H), F32),
    )(edge_attr, enc_e['W1'], rs(enc_e['b1']), enc_e['W2'],
      rs(enc_e['b2']), rs(enc_e['g']), rs(enc_e['bln']))

    gather_k = _make_gather(n_nodes, n_edges)
    scatter_k = _make_scatter(n_nodes, n_edges)

    n_steps = len(mp)
    for i in range(n_steps):
        blk = mp[i]
        ew = blk['edge']
        nw = blk['node']
        s = gather_k(p_tab, q_tab, src3, dst3)
        e = pl.pallas_call(
            _edgestep_body,
            grid=(ge,),
            in_specs=[_row(be, H), _row(be, H), _bcast(H, H), _bcast(1, H),
                      _bcast(H, H), _bcast(1, H), _bcast(1, H), _bcast(1, H)],
            out_specs=_row(be, H),
            out_shape=jax.ShapeDtypeStruct((n_edges, H), F32),
        )(e, s, ew['W1'][:H], rs(ew['b1']), ew['W2'], rs(ew['b2']),
          rs(ew['g']), rs(ew['bln']))
        parts = scatter_k(e, dst3)
        p0, p1 = parts[0], parts[1]
        if i + 1 < n_steps:
            w1s = mp[i + 1]['edge']['W1'][H:2 * H]
            w1d = mp[i + 1]['edge']['W1'][2 * H:]
            h, p_tab, q_tab = pl.pallas_call(
                _nodestep_body,
                grid=(gn,),
                in_specs=[_row(bn, H), _row(bn, H), _row(bn, H),
                          _bcast(H, H), _bcast(H, H), _bcast(1, H),
                          _bcast(H, H), _bcast(1, H), _bcast(1, H),
                          _bcast(1, H), _bcast(H, H), _bcast(H, H)],
                out_specs=(_row(bn, H),) * 3,
                out_shape=(jax.ShapeDtypeStruct((n_nodes, H), F32),) * 3,
            )(h, p0, p1, nw['W1'][:H], nw['W1'][H:], rs(nw['b1']),
              nw['W2'], rs(nw['b2']), rs(nw['g']), rs(nw['bln']), w1s, w1d)
        else:
            h = pl.pallas_call(
                _nodelast_body,
                grid=(gn,),
                in_specs=[_row(bn, H), _row(bn, H), _row(bn, H),
                          _bcast(H, H), _bcast(H, H), _bcast(1, H),
                          _bcast(H, H), _bcast(1, H), _bcast(1, H),
                          _bcast(1, H)],
                out_specs=_row(bn, H),
                out_shape=jax.ShapeDtypeStruct((n_nodes, H), F32),
            )(h, p0, p1, nw['W1'][:H], nw['W1'][H:], rs(nw['b1']),
              nw['W2'], rs(nw['b2']), rs(nw['g']), rs(nw['bln']))

    pred = pl.pallas_call(
        _dec_body,
        grid=(gn,),
        in_specs=[_row(bn, H), _bcast(H, H), _bcast(1, H), _bcast(H, 4),
                  _bcast(1, 4)],
        out_specs=_row(bn, 4),
        out_shape=jax.ShapeDtypeStruct((n_nodes, 4), F32),
    )(h, dec['W1'], rs(dec['b1']), dec['W2'], rs(dec['b2']))

    return (pred, tgt)
